# baseline (speedup 1.0000x reference)
import functools

import jax
import jax.numpy as jnp
from jax import lax
from jax.experimental import pallas as pl
from jax.experimental.pallas import tpu as pltpu
from jax.experimental.pallas import tpu_sc as plsc

B = 16384
UDIM = 64
MDIM = 64
H1 = 256
H2 = 64

NC = 2
NS = 16
NW = NC * NS
ROWS_PER_W = B // NW
CHUNK = 128
NCHUNK = ROWS_PER_W // CHUNK


def _sc_gather(u_idx, m_idx, user_table, movie_table):
    mesh = plsc.VectorSubcoreMesh(core_axis_name="c", subcore_axis_name="s")

    @functools.partial(
        pl.kernel,
        out_type=(
            jax.ShapeDtypeStruct((B, UDIM), jnp.float32),
            jax.ShapeDtypeStruct((B, MDIM), jnp.float32),
        ),
        mesh=mesh,
        compiler_params=pltpu.CompilerParams(use_tc_tiling_on_sc=False),
        scratch_types=[
            pltpu.VMEM((NCHUNK, CHUNK), jnp.int32),
            pltpu.VMEM((NCHUNK, CHUNK), jnp.int32),
            pltpu.VMEM((ROWS_PER_W, UDIM), jnp.float32),
            pltpu.VMEM((ROWS_PER_W, MDIM), jnp.float32),
            pltpu.SemaphoreType.DMA,
        ],
    )
    def gather_kernel(uidx_hbm, midx_hbm, utab_hbm, mtab_hbm,
                      uout_hbm, mout_hbm,
                      uidx_v, midx_v, urows_v, mrows_v, sem):
        wid = lax.axis_index("s") * NC + lax.axis_index("c")
        base = wid * ROWS_PER_W
        pltpu.sync_copy(uidx_hbm.at[wid], uidx_v)
        pltpu.sync_copy(midx_hbm.at[wid], midx_v)
        copies = []
        for j in range(NCHUNK):
            copies.append(pltpu.async_copy(
                utab_hbm.at[uidx_v.at[j]],
                urows_v.at[pl.ds(j * CHUNK, CHUNK)], sem))
            copies.append(pltpu.async_copy(
                mtab_hbm.at[midx_v.at[j]],
                mrows_v.at[pl.ds(j * CHUNK, CHUNK)], sem))
        for c in copies:
            c.wait()
        pltpu.sync_copy(urows_v, uout_hbm.at[pl.ds(base, ROWS_PER_W)])
        pltpu.sync_copy(mrows_v, mout_hbm.at[pl.ds(base, ROWS_PER_W)])

    return gather_kernel(u_idx, m_idx, user_table, movie_table)


def _tc_mlp(ue, me, W1, b1, W2, b2, W3, b3):
    Wa = W1[:UDIM]
    Wb = W1[UDIM:]
    BLK = 4096

    def mlp_kernel(ue_ref, me_ref, wa_ref, wb_ref, b1_ref,
                   w2_ref, b2_ref, w3_ref, b3_ref, o_ref):
        h = jnp.dot(ue_ref[...], wa_ref[...], preferred_element_type=jnp.float32)
        h = h + jnp.dot(me_ref[...], wb_ref[...], preferred_element_type=jnp.float32)
        h = jnp.maximum(h + b1_ref[...], 0.0)
        h = jnp.dot(h, w2_ref[...], preferred_element_type=jnp.float32)
        h = jnp.maximum(h + b2_ref[...], 0.0)
        o_ref[...] = (jnp.dot(h, w3_ref[...], preferred_element_type=jnp.float32)
                      + b3_ref[...])

    return pl.pallas_call(
        mlp_kernel,
        grid=(B // BLK,),
        in_specs=[
            pl.BlockSpec((BLK, UDIM), lambda i: (i, 0)),
            pl.BlockSpec((BLK, MDIM), lambda i: (i, 0)),
            pl.BlockSpec((UDIM, H1), lambda i: (0, 0)),
            pl.BlockSpec((MDIM, H1), lambda i: (0, 0)),
            pl.BlockSpec((1, H1), lambda i: (0, 0)),
            pl.BlockSpec((H1, H2), lambda i: (0, 0)),
            pl.BlockSpec((1, H2), lambda i: (0, 0)),
            pl.BlockSpec((H2, 1), lambda i: (0, 0)),
            pl.BlockSpec((1, 1), lambda i: (0, 0)),
        ],
        out_specs=pl.BlockSpec((BLK, 1), lambda i: (i, 0)),
        out_shape=jax.ShapeDtypeStruct((B, 1), jnp.float32),
    )(ue, me, Wa, Wb, b1.reshape(1, H1), W2, b2.reshape(1, H2),
      W3, b3.reshape(1, 1))


def kernel(user_id, movie_title, user_table, movie_table,
           W1, b1, W2, b2, W3, b3):
    u_idx = user_id.astype(jnp.int32).reshape(NW, NCHUNK, CHUNK)
    m_idx = movie_title.astype(jnp.int32).reshape(NW, NCHUNK, CHUNK)
    ue, me = _sc_gather(u_idx, m_idx, user_table, movie_table)
    return _tc_mlp(ue, me, W1, b1, W2, b2, W3, b3)
